# baseline (speedup 1.0000x reference)
import jax
import jax.numpy as jnp
from jax.experimental import pallas as pl

_K = 128
_INT_MIN = -2147483648
_INT_MAX = 2147483647


def _mask_kernel(reps_ref, mask_ref, bin_ref):
    x = reps_ref[...]
    rows, n = x.shape
    b = jax.lax.bitcast_convert_type(x, jnp.int32)
    key = jnp.where(b >= 0, b, (~b) ^ jnp.int32(_INT_MIN))

    def hybrid_step(i, state):
        lo, hi, done, thr, f_lo, f_hi, side = state
        x_xor = lo ^ hi
        mid_b = (lo & hi) + (x_xor >> 1) + (x_xor & 1)
        lo_f = lo.astype(jnp.float32)
        span = hi.astype(jnp.float32) + 1.0 - lo_f
        frac = f_lo / jnp.maximum(f_lo - f_hi, 1.0)
        mid_f = jnp.clip(lo_f + frac * span, -2.14e9, 2.14e9)
        mid_s = jnp.clip(mid_f.astype(jnp.int32), lo + 1, hi)
        mid = jnp.where(i >= 4, mid_s, mid_b)
        ones_col = jnp.ones((n, 8), jnp.bfloat16)
        cnt = jax.lax.dot_general(
            (key >= mid).astype(jnp.bfloat16), ones_col,
            (((1,), (0,)), ((), ())),
            preferred_element_type=jnp.float32)[:, :1]
        res = cnt - jnp.float32(_K)
        ge = cnt >= _K
        hit = (cnt == _K) & (done < 1)
        thr = jnp.where(hit, mid, thr)
        done = jnp.where(hit, jnp.int32(1), done)
        lo = jnp.where(ge, mid, lo)
        hi = jnp.where(ge, hi, mid - 1)
        stall_hi = jnp.logical_not(ge) & (side < 0)
        stall_lo = ge & (side > 0)
        f_lo = jnp.where(ge, res, jnp.where(stall_hi, f_lo * 0.5, f_lo))
        f_hi = jnp.where(ge, jnp.where(stall_lo, f_hi * 0.5, f_hi), res)
        side = jnp.where(ge, jnp.int32(1), jnp.int32(-1))
        return lo, hi, done, thr, f_lo, f_hi, side

    lo0 = jnp.full((rows, 1), _INT_MIN, jnp.int32)
    hi0 = jnp.full((rows, 1), _INT_MAX, jnp.int32)
    done0 = jnp.zeros((rows, 1), jnp.int32)
    thr0 = jnp.zeros((rows, 1), jnp.int32)
    flo0 = jnp.full((rows, 1), float(n - _K), jnp.float32)
    fhi0 = jnp.full((rows, 1), float(-_K), jnp.float32)
    side0 = jnp.zeros((rows, 1), jnp.int32)
    lo, hi, done, thr, _, _, _ = jax.lax.fori_loop(
        0, 22, hybrid_step, (lo0, hi0, done0, thr0, flo0, fhi0, side0))

    def mop_cond(state):
        i, lo, hi, done = state[0], state[1], state[2], state[3]
        resolved = (done > 0) | (lo >= hi)
        return (i < 40) & jnp.logical_not(jnp.all(resolved))

    def mop_step(state):
        i, lo, hi, done, thr = state
        x_xor = lo ^ hi
        mid = (lo & hi) + (x_xor >> 1) + (x_xor & 1)
        cnt = jnp.sum((key >= mid).astype(jnp.int32), axis=1, keepdims=True)
        ge = cnt >= _K
        hit = (cnt == _K) & (done < 1)
        thr = jnp.where(hit, mid, thr)
        done = jnp.where(hit, jnp.int32(1), done)
        lo = jnp.where(ge, mid, lo)
        hi = jnp.where(ge, hi, mid - 1)
        return i + 1, lo, hi, done, thr

    _, lo, hi, done, thr = jax.lax.while_loop(
        mop_cond, mop_step, (jnp.int32(0), lo, hi, done, thr))
    done = done > 0

    on_clean = (key >= thr).astype(jnp.float32)

    def tie_path(_):
        t_exact = lo
        c_gt = jnp.sum((key > t_exact).astype(jnp.int32), axis=1,
                       keepdims=True)
        need = _K - c_gt

        def index_step(_, state):
            lo_i, hi_i = state
            mid = (lo_i & hi_i) + ((lo_i ^ hi_i) >> 1)
            iota = jax.lax.broadcasted_iota(jnp.int32, (rows, n), 1)
            sel = (key == t_exact) & (iota <= mid)
            cnt = jnp.sum(sel.astype(jnp.int32), axis=1, keepdims=True)
            ge = cnt >= need
            return jnp.where(ge, lo_i, mid + 1), jnp.where(ge, mid, hi_i)

        lo0i = jnp.zeros((rows, 1), jnp.int32)
        hi0i = jnp.full((rows, 1), n - 1, jnp.int32)
        cut, _ = jax.lax.fori_loop(0, 13, index_step, (lo0i, hi0i))
        iota = jax.lax.broadcasted_iota(jnp.int32, (rows, n), 1)
        on_tie = ((key > t_exact)
                  | ((key == t_exact) & (iota <= cut))).astype(jnp.float32)
        return jnp.where(done, on_clean, on_tie)

    on = jax.lax.cond(jnp.all(done),
                      lambda _: on_clean,
                      tie_path,
                      None)
    mask_ref[...] = on * jnp.float32(2.0) - jnp.float32(1.0)
    bin_ref[...] = on.astype(jnp.bfloat16)


def _overlap_kernel(bi_ref, bj_ref, out_ref):
    out_ref[...] = jax.lax.dot_general(
        bi_ref[...], bj_ref[...], (((1,), (1,)), ((), ())),
        preferred_element_type=jnp.float32)


def kernel(reps):
    m, n = reps.shape
    rows = 128
    mask, binary = pl.pallas_call(
        _mask_kernel,
        grid=(m // rows,),
        in_specs=[pl.BlockSpec((rows, n), lambda i: (i, 0))],
        out_specs=[
            pl.BlockSpec((rows, n), lambda i: (i, 0)),
            pl.BlockSpec((rows, n), lambda i: (i, 0)),
        ],
        out_shape=[
            jax.ShapeDtypeStruct((m, n), jnp.float32),
            jax.ShapeDtypeStruct((m, n), jnp.bfloat16),
        ],
    )(reps)

    bm = 512
    overlaps = pl.pallas_call(
        _overlap_kernel,
        grid=(m // bm, m // bm),
        in_specs=[
            pl.BlockSpec((bm, n), lambda i, j: (i, 0)),
            pl.BlockSpec((bm, n), lambda i, j: (j, 0)),
        ],
        out_specs=pl.BlockSpec((bm, bm), lambda i, j: (i, j)),
        out_shape=jax.ShapeDtypeStruct((m, m), jnp.float32),
    )(binary, binary)
    return (mask, overlaps)


# speedup vs baseline: 1.3388x; 1.3388x over previous
import jax
import jax.numpy as jnp
from jax.experimental import pallas as pl

_K = 128
_INT_MIN = -2147483648
_INT_MAX = 2147483647


def _mask_kernel(reps_ref, mask_ref, bin_ref):
    x = reps_ref[...]
    rows, n = x.shape
    b = jax.lax.bitcast_convert_type(x, jnp.int32)
    key = jnp.where(b >= 0, b, (~b) ^ jnp.int32(_INT_MIN))

    def hybrid_step(i, state):
        lo, hi, done, thr, f_lo, f_hi, side = state
        lo_f = lo.astype(jnp.float32)
        span = hi.astype(jnp.float32) + 1.0 - lo_f
        frac = f_lo / jnp.maximum(f_lo - f_hi, 1.0)
        mid_f = jnp.clip(lo_f + frac * span, -2.14e9, 2.14e9)
        mid = jnp.clip(mid_f.astype(jnp.int32), lo + 1, hi)
        cnt = jnp.sum((key >= mid).astype(jnp.int32), axis=1, keepdims=True)
        res = cnt.astype(jnp.float32) - jnp.float32(_K)
        ge = cnt >= _K
        hit = (cnt == _K) & (done < 1)
        thr = jnp.where(hit, mid, thr)
        done = jnp.where(hit, jnp.int32(1), done)
        lo = jnp.where(ge, mid, lo)
        hi = jnp.where(ge, hi, mid - 1)
        stall_hi = jnp.logical_not(ge) & (side < 0)
        stall_lo = ge & (side > 0)
        f_lo = jnp.where(ge, res, jnp.where(stall_hi, f_lo * 0.5, f_lo))
        f_hi = jnp.where(ge, jnp.where(stall_lo, f_hi * 0.5, f_hi), res)
        side = jnp.where(ge, jnp.int32(1), jnp.int32(-1))
        return lo, hi, done, thr, f_lo, f_hi, side

    lo0 = jnp.min(key, axis=1, keepdims=True)
    hi0 = jnp.max(key, axis=1, keepdims=True)
    done0 = jnp.zeros((rows, 1), jnp.int32)
    thr0 = jnp.zeros((rows, 1), jnp.int32)
    flo0 = jnp.full((rows, 1), float(n - _K), jnp.float32)
    fhi0 = jnp.full((rows, 1), float(-_K), jnp.float32)
    side0 = jnp.zeros((rows, 1), jnp.int32)
    lo, hi, done, thr, _, _, _ = jax.lax.fori_loop(
        0, 18, hybrid_step, (lo0, hi0, done0, thr0, flo0, fhi0, side0))

    def mop_cond(state):
        i, lo, hi, done = state[0], state[1], state[2], state[3]
        resolved = (done > 0) | (lo >= hi)
        return (i < 40) & jnp.logical_not(jnp.all(resolved))

    def mop_step(state):
        i, lo, hi, done, thr = state
        x_xor = lo ^ hi
        mid = (lo & hi) + (x_xor >> 1) + (x_xor & 1)
        cnt = jnp.sum((key >= mid).astype(jnp.int32), axis=1, keepdims=True)
        ge = cnt >= _K
        hit = (cnt == _K) & (done < 1)
        thr = jnp.where(hit, mid, thr)
        done = jnp.where(hit, jnp.int32(1), done)
        lo = jnp.where(ge, mid, lo)
        hi = jnp.where(ge, hi, mid - 1)
        return i + 1, lo, hi, done, thr

    _, lo, hi, done, thr = jax.lax.while_loop(
        mop_cond, mop_step, (jnp.int32(0), lo, hi, done, thr))
    done = done > 0

    on_clean = (key >= thr).astype(jnp.float32)

    def tie_path(_):
        t_exact = lo
        c_gt = jnp.sum((key > t_exact).astype(jnp.int32), axis=1,
                       keepdims=True)
        need = _K - c_gt

        def index_step(_, state):
            lo_i, hi_i = state
            mid = (lo_i & hi_i) + ((lo_i ^ hi_i) >> 1)
            iota = jax.lax.broadcasted_iota(jnp.int32, (rows, n), 1)
            sel = (key == t_exact) & (iota <= mid)
            cnt = jnp.sum(sel.astype(jnp.int32), axis=1, keepdims=True)
            ge = cnt >= need
            return jnp.where(ge, lo_i, mid + 1), jnp.where(ge, mid, hi_i)

        lo0i = jnp.zeros((rows, 1), jnp.int32)
        hi0i = jnp.full((rows, 1), n - 1, jnp.int32)
        cut, _ = jax.lax.fori_loop(0, 13, index_step, (lo0i, hi0i))
        iota = jax.lax.broadcasted_iota(jnp.int32, (rows, n), 1)
        on_tie = ((key > t_exact)
                  | ((key == t_exact) & (iota <= cut))).astype(jnp.float32)
        return jnp.where(done, on_clean, on_tie)

    on = jax.lax.cond(jnp.all(done),
                      lambda _: on_clean,
                      tie_path,
                      None)
    mask_ref[...] = on * jnp.float32(2.0) - jnp.float32(1.0)
    bin_ref[...] = on.astype(jnp.bfloat16)


def _overlap_kernel(bi_ref, bj_ref, out_ref):
    out_ref[...] = jax.lax.dot_general(
        bi_ref[...], bj_ref[...], (((1,), (1,)), ((), ())),
        preferred_element_type=jnp.float32)


def kernel(reps):
    m, n = reps.shape
    rows = 128
    mask, binary = pl.pallas_call(
        _mask_kernel,
        grid=(m // rows,),
        in_specs=[pl.BlockSpec((rows, n), lambda i: (i, 0))],
        out_specs=[
            pl.BlockSpec((rows, n), lambda i: (i, 0)),
            pl.BlockSpec((rows, n), lambda i: (i, 0)),
        ],
        out_shape=[
            jax.ShapeDtypeStruct((m, n), jnp.float32),
            jax.ShapeDtypeStruct((m, n), jnp.bfloat16),
        ],
    )(reps)

    bm = 512
    overlaps = pl.pallas_call(
        _overlap_kernel,
        grid=(m // bm, m // bm),
        in_specs=[
            pl.BlockSpec((bm, n), lambda i, j: (i, 0)),
            pl.BlockSpec((bm, n), lambda i, j: (j, 0)),
        ],
        out_specs=pl.BlockSpec((bm, bm), lambda i, j: (i, j)),
        out_shape=jax.ShapeDtypeStruct((m, m), jnp.float32),
    )(binary, binary)
    return (mask, overlaps)


# speedup vs baseline: 1.4032x; 1.0481x over previous
import jax
import jax.numpy as jnp
from jax.experimental import pallas as pl

_K = 128
_INT_MIN = -2147483648
_INT_MAX = 2147483647


def _mask_kernel(reps_ref, mask_ref, bin_ref):
    x = reps_ref[...]
    rows, n = x.shape
    b = jax.lax.bitcast_convert_type(x, jnp.int32)
    key = jnp.where(b >= 0, b, (~b) ^ jnp.int32(_INT_MIN))

    def hybrid_step(i, state):
        lo, hi, done, thr, f_lo, f_hi, side = state
        lo_f = lo.astype(jnp.float32)
        span = hi.astype(jnp.float32) + 1.0 - lo_f
        frac = f_lo / jnp.maximum(f_lo - f_hi, 1.0)
        mid_f = jnp.clip(lo_f + frac * span, -2.14e9, 2.14e9)
        mid = jnp.clip(mid_f.astype(jnp.int32), lo + 1, hi)
        cnt = jnp.sum((key >= mid).astype(jnp.int32), axis=1, keepdims=True)
        res = cnt.astype(jnp.float32) - jnp.float32(_K)
        ge = cnt >= _K
        hit = (cnt == _K) & (done < 1)
        thr = jnp.where(hit, mid, thr)
        done = jnp.where(hit, jnp.int32(1), done)
        lo = jnp.where(ge, mid, lo)
        hi = jnp.where(ge, hi, mid - 1)
        stall_hi = jnp.logical_not(ge) & (side < 0)
        stall_lo = ge & (side > 0)
        f_lo = jnp.where(ge, res, jnp.where(stall_hi, f_lo * 0.5, f_lo))
        f_hi = jnp.where(ge, jnp.where(stall_lo, f_hi * 0.5, f_hi), res)
        side = jnp.where(ge, jnp.int32(1), jnp.int32(-1))
        return lo, hi, done, thr, f_lo, f_hi, side

    lo0 = jnp.min(key, axis=1, keepdims=True)
    hi0 = jnp.max(key, axis=1, keepdims=True)
    done0 = jnp.zeros((rows, 1), jnp.int32)
    thr0 = jnp.zeros((rows, 1), jnp.int32)
    flo0 = jnp.full((rows, 1), float(n - _K), jnp.float32)
    fhi0 = jnp.full((rows, 1), float(-_K), jnp.float32)
    side0 = jnp.zeros((rows, 1), jnp.int32)
    lo, hi, done, thr, _, _, _ = jax.lax.fori_loop(
        0, 16, hybrid_step, (lo0, hi0, done0, thr0, flo0, fhi0, side0))

    def mop_cond(state):
        i, lo, hi, done = state[0], state[1], state[2], state[3]
        resolved = (done > 0) | (lo >= hi)
        return (i < 40) & jnp.logical_not(jnp.all(resolved))

    def mop_step(state):
        i, lo, hi, done, thr = state
        x_xor = lo ^ hi
        mid = (lo & hi) + (x_xor >> 1) + (x_xor & 1)
        cnt = jnp.sum((key >= mid).astype(jnp.int32), axis=1, keepdims=True)
        ge = cnt >= _K
        hit = (cnt == _K) & (done < 1)
        thr = jnp.where(hit, mid, thr)
        done = jnp.where(hit, jnp.int32(1), done)
        lo = jnp.where(ge, mid, lo)
        hi = jnp.where(ge, hi, mid - 1)
        return i + 1, lo, hi, done, thr

    _, lo, hi, done, thr = jax.lax.while_loop(
        mop_cond, mop_step, (jnp.int32(0), lo, hi, done, thr))
    done = done > 0

    on_clean = (key >= thr).astype(jnp.float32)

    def tie_path(_):
        t_exact = lo
        c_gt = jnp.sum((key > t_exact).astype(jnp.int32), axis=1,
                       keepdims=True)
        need = _K - c_gt

        def index_step(_, state):
            lo_i, hi_i = state
            mid = (lo_i & hi_i) + ((lo_i ^ hi_i) >> 1)
            iota = jax.lax.broadcasted_iota(jnp.int32, (rows, n), 1)
            sel = (key == t_exact) & (iota <= mid)
            cnt = jnp.sum(sel.astype(jnp.int32), axis=1, keepdims=True)
            ge = cnt >= need
            return jnp.where(ge, lo_i, mid + 1), jnp.where(ge, mid, hi_i)

        lo0i = jnp.zeros((rows, 1), jnp.int32)
        hi0i = jnp.full((rows, 1), n - 1, jnp.int32)
        cut, _ = jax.lax.fori_loop(0, 13, index_step, (lo0i, hi0i))
        iota = jax.lax.broadcasted_iota(jnp.int32, (rows, n), 1)
        on_tie = ((key > t_exact)
                  | ((key == t_exact) & (iota <= cut))).astype(jnp.float32)
        return jnp.where(done, on_clean, on_tie)

    on = jax.lax.cond(jnp.all(done),
                      lambda _: on_clean,
                      tie_path,
                      None)
    mask_ref[...] = on * jnp.float32(2.0) - jnp.float32(1.0)
    bin_ref[...] = on.astype(jnp.bfloat16)


def _overlap_kernel(bi_ref, bj_ref, out_ref):
    out_ref[...] = jax.lax.dot_general(
        bi_ref[...], bj_ref[...], (((1,), (1,)), ((), ())),
        preferred_element_type=jnp.float32)


def kernel(reps):
    m, n = reps.shape
    rows = 128
    mask, binary = pl.pallas_call(
        _mask_kernel,
        grid=(m // rows,),
        in_specs=[pl.BlockSpec((rows, n), lambda i: (i, 0))],
        out_specs=[
            pl.BlockSpec((rows, n), lambda i: (i, 0)),
            pl.BlockSpec((rows, n), lambda i: (i, 0)),
        ],
        out_shape=[
            jax.ShapeDtypeStruct((m, n), jnp.float32),
            jax.ShapeDtypeStruct((m, n), jnp.bfloat16),
        ],
    )(reps)

    bm = 512
    overlaps = pl.pallas_call(
        _overlap_kernel,
        grid=(m // bm, m // bm),
        in_specs=[
            pl.BlockSpec((bm, n), lambda i, j: (i, 0)),
            pl.BlockSpec((bm, n), lambda i, j: (j, 0)),
        ],
        out_specs=pl.BlockSpec((bm, bm), lambda i, j: (i, j)),
        out_shape=jax.ShapeDtypeStruct((m, m), jnp.float32),
    )(binary, binary)
    return (mask, overlaps)


# speedup vs baseline: 1.4208x; 1.0126x over previous
import jax
import jax.numpy as jnp
from jax.experimental import pallas as pl

_K = 128
_INT_MIN = -2147483648
_INT_MAX = 2147483647


def _mask_kernel(reps_ref, mask_ref, bin_ref):
    x = reps_ref[...]
    rows, n = x.shape
    b = jax.lax.bitcast_convert_type(x, jnp.int32)
    key = jnp.where(b >= 0, b, (~b) ^ jnp.int32(_INT_MIN))

    def hybrid_step(i, state):
        lo, hi, done, thr, f_lo, f_hi, side = state
        lo_f = lo.astype(jnp.float32)
        span = hi.astype(jnp.float32) + 1.0 - lo_f
        frac = f_lo / jnp.maximum(f_lo - f_hi, 1.0)
        mid_f = jnp.clip(lo_f + frac * span, -2.14e9, 2.14e9)
        mid = jnp.clip(mid_f.astype(jnp.int32), lo + 1, hi)
        cnt = jnp.sum((key >= mid).astype(jnp.int32), axis=1, keepdims=True)
        res = cnt.astype(jnp.float32) - jnp.float32(_K)
        ge = cnt >= _K
        hit = (cnt == _K) & (done < 1)
        thr = jnp.where(hit, mid, thr)
        done = jnp.where(hit, jnp.int32(1), done)
        lo = jnp.where(ge, mid, lo)
        hi = jnp.where(ge, hi, mid - 1)
        stall_hi = jnp.logical_not(ge) & (side < 0)
        stall_lo = ge & (side > 0)
        f_lo = jnp.where(ge, res, jnp.where(stall_hi, f_lo * 0.5, f_lo))
        f_hi = jnp.where(ge, jnp.where(stall_lo, f_hi * 0.5, f_hi), res)
        side = jnp.where(ge, jnp.int32(1), jnp.int32(-1))
        return lo, hi, done, thr, f_lo, f_hi, side

    lo0 = jnp.min(key, axis=1, keepdims=True)
    hi0 = jnp.max(key, axis=1, keepdims=True)
    done0 = jnp.zeros((rows, 1), jnp.int32)
    thr0 = jnp.zeros((rows, 1), jnp.int32)
    flo0 = jnp.full((rows, 1), float(n - _K), jnp.float32)
    fhi0 = jnp.full((rows, 1), float(-_K), jnp.float32)
    side0 = jnp.zeros((rows, 1), jnp.int32)
    lo, hi, done, thr, _, _, _ = jax.lax.fori_loop(
        0, 13, hybrid_step, (lo0, hi0, done0, thr0, flo0, fhi0, side0))

    def mop_cond(state):
        i, lo, hi, done = state[0], state[1], state[2], state[3]
        resolved = (done > 0) | (lo >= hi)
        return (i < 40) & jnp.logical_not(jnp.all(resolved))

    def mop_step(state):
        i, lo, hi, done, thr = state
        x_xor = lo ^ hi
        mid = (lo & hi) + (x_xor >> 1) + (x_xor & 1)
        cnt = jnp.sum((key >= mid).astype(jnp.int32), axis=1, keepdims=True)
        ge = cnt >= _K
        hit = (cnt == _K) & (done < 1)
        thr = jnp.where(hit, mid, thr)
        done = jnp.where(hit, jnp.int32(1), done)
        lo = jnp.where(ge, mid, lo)
        hi = jnp.where(ge, hi, mid - 1)
        return i + 1, lo, hi, done, thr

    _, lo, hi, done, thr = jax.lax.while_loop(
        mop_cond, mop_step, (jnp.int32(0), lo, hi, done, thr))
    done = done > 0

    on_clean = (key >= thr).astype(jnp.float32)

    def tie_path(_):
        t_exact = lo
        c_gt = jnp.sum((key > t_exact).astype(jnp.int32), axis=1,
                       keepdims=True)
        need = _K - c_gt

        def index_step(_, state):
            lo_i, hi_i = state
            mid = (lo_i & hi_i) + ((lo_i ^ hi_i) >> 1)
            iota = jax.lax.broadcasted_iota(jnp.int32, (rows, n), 1)
            sel = (key == t_exact) & (iota <= mid)
            cnt = jnp.sum(sel.astype(jnp.int32), axis=1, keepdims=True)
            ge = cnt >= need
            return jnp.where(ge, lo_i, mid + 1), jnp.where(ge, mid, hi_i)

        lo0i = jnp.zeros((rows, 1), jnp.int32)
        hi0i = jnp.full((rows, 1), n - 1, jnp.int32)
        cut, _ = jax.lax.fori_loop(0, 13, index_step, (lo0i, hi0i))
        iota = jax.lax.broadcasted_iota(jnp.int32, (rows, n), 1)
        on_tie = ((key > t_exact)
                  | ((key == t_exact) & (iota <= cut))).astype(jnp.float32)
        return jnp.where(done, on_clean, on_tie)

    on = jax.lax.cond(jnp.all(done),
                      lambda _: on_clean,
                      tie_path,
                      None)
    mask_ref[...] = on * jnp.float32(2.0) - jnp.float32(1.0)
    bin_ref[...] = on.astype(jnp.bfloat16)


def _overlap_kernel(bi_ref, bj_ref, out_ref):
    out_ref[...] = jax.lax.dot_general(
        bi_ref[...], bj_ref[...], (((1,), (1,)), ((), ())),
        preferred_element_type=jnp.float32)


def kernel(reps):
    m, n = reps.shape
    rows = 128
    mask, binary = pl.pallas_call(
        _mask_kernel,
        grid=(m // rows,),
        in_specs=[pl.BlockSpec((rows, n), lambda i: (i, 0))],
        out_specs=[
            pl.BlockSpec((rows, n), lambda i: (i, 0)),
            pl.BlockSpec((rows, n), lambda i: (i, 0)),
        ],
        out_shape=[
            jax.ShapeDtypeStruct((m, n), jnp.float32),
            jax.ShapeDtypeStruct((m, n), jnp.bfloat16),
        ],
    )(reps)

    bm = 512
    overlaps = pl.pallas_call(
        _overlap_kernel,
        grid=(m // bm, m // bm),
        in_specs=[
            pl.BlockSpec((bm, n), lambda i, j: (i, 0)),
            pl.BlockSpec((bm, n), lambda i, j: (j, 0)),
        ],
        out_specs=pl.BlockSpec((bm, bm), lambda i, j: (i, j)),
        out_shape=jax.ShapeDtypeStruct((m, m), jnp.float32),
    )(binary, binary)
    return (mask, overlaps)


# speedup vs baseline: 1.4224x; 1.0011x over previous
import jax
import jax.numpy as jnp
from jax.experimental import pallas as pl

_K = 128
_INT_MIN = -2147483648
_INT_MAX = 2147483647


def _mask_kernel(reps_ref, mask_ref, bin_ref):
    x = reps_ref[...]
    rows, n = x.shape
    b = jax.lax.bitcast_convert_type(x, jnp.int32)
    key = jnp.where(b >= 0, b, (~b) ^ jnp.int32(_INT_MIN))

    def hybrid_step(i, state):
        lo, hi, done, thr, f_lo, f_hi, side = state
        lo_f = lo.astype(jnp.float32)
        span = hi.astype(jnp.float32) + 1.0 - lo_f
        frac = f_lo / jnp.maximum(f_lo - f_hi, 1.0)
        mid_f = jnp.clip(lo_f + frac * span, -2.14e9, 2.14e9)
        mid = jnp.clip(mid_f.astype(jnp.int32), lo + 1, hi)
        cnt = jnp.sum((key >= mid).astype(jnp.int32), axis=1, keepdims=True)
        res = cnt.astype(jnp.float32) - jnp.float32(_K)
        ge = cnt >= _K
        hit = (cnt == _K) & (done < 1)
        thr = jnp.where(hit, mid, thr)
        done = jnp.where(hit, jnp.int32(1), done)
        lo = jnp.where(ge, mid, lo)
        hi = jnp.where(ge, hi, mid - 1)
        stall_hi = jnp.logical_not(ge) & (side < 0)
        stall_lo = ge & (side > 0)
        f_lo = jnp.where(ge, res, jnp.where(stall_hi, f_lo * 0.5, f_lo))
        f_hi = jnp.where(ge, jnp.where(stall_lo, f_hi * 0.5, f_hi), res)
        side = jnp.where(ge, jnp.int32(1), jnp.int32(-1))
        return lo, hi, done, thr, f_lo, f_hi, side

    lo0 = jnp.min(key, axis=1, keepdims=True)
    hi0 = jnp.max(key, axis=1, keepdims=True)
    done0 = jnp.zeros((rows, 1), jnp.int32)
    thr0 = jnp.zeros((rows, 1), jnp.int32)
    flo0 = jnp.full((rows, 1), float(n - _K), jnp.float32)
    fhi0 = jnp.full((rows, 1), float(-_K), jnp.float32)
    side0 = jnp.zeros((rows, 1), jnp.int32)
    lo, hi, done, thr, _, _, _ = jax.lax.fori_loop(
        0, 11, hybrid_step, (lo0, hi0, done0, thr0, flo0, fhi0, side0))

    def mop_cond(state):
        i, lo, hi, done = state[0], state[1], state[2], state[3]
        resolved = (done > 0) | (lo >= hi)
        return (i < 40) & jnp.logical_not(jnp.all(resolved))

    def mop_step(state):
        i, lo, hi, done, thr = state
        x_xor = lo ^ hi
        mid = (lo & hi) + (x_xor >> 1) + (x_xor & 1)
        cnt = jnp.sum((key >= mid).astype(jnp.int32), axis=1, keepdims=True)
        ge = cnt >= _K
        hit = (cnt == _K) & (done < 1)
        thr = jnp.where(hit, mid, thr)
        done = jnp.where(hit, jnp.int32(1), done)
        lo = jnp.where(ge, mid, lo)
        hi = jnp.where(ge, hi, mid - 1)
        return i + 1, lo, hi, done, thr

    _, lo, hi, done, thr = jax.lax.while_loop(
        mop_cond, mop_step, (jnp.int32(0), lo, hi, done, thr))
    done = done > 0

    on_clean = (key >= thr).astype(jnp.float32)

    def tie_path(_):
        t_exact = lo
        c_gt = jnp.sum((key > t_exact).astype(jnp.int32), axis=1,
                       keepdims=True)
        need = _K - c_gt

        def index_step(_, state):
            lo_i, hi_i = state
            mid = (lo_i & hi_i) + ((lo_i ^ hi_i) >> 1)
            iota = jax.lax.broadcasted_iota(jnp.int32, (rows, n), 1)
            sel = (key == t_exact) & (iota <= mid)
            cnt = jnp.sum(sel.astype(jnp.int32), axis=1, keepdims=True)
            ge = cnt >= need
            return jnp.where(ge, lo_i, mid + 1), jnp.where(ge, mid, hi_i)

        lo0i = jnp.zeros((rows, 1), jnp.int32)
        hi0i = jnp.full((rows, 1), n - 1, jnp.int32)
        cut, _ = jax.lax.fori_loop(0, 13, index_step, (lo0i, hi0i))
        iota = jax.lax.broadcasted_iota(jnp.int32, (rows, n), 1)
        on_tie = ((key > t_exact)
                  | ((key == t_exact) & (iota <= cut))).astype(jnp.float32)
        return jnp.where(done, on_clean, on_tie)

    on = jax.lax.cond(jnp.all(done),
                      lambda _: on_clean,
                      tie_path,
                      None)
    mask_ref[...] = on * jnp.float32(2.0) - jnp.float32(1.0)
    bin_ref[...] = on.astype(jnp.bfloat16)


def _overlap_kernel(bi_ref, bj_ref, out_ref):
    out_ref[...] = jax.lax.dot_general(
        bi_ref[...], bj_ref[...], (((1,), (1,)), ((), ())),
        preferred_element_type=jnp.float32)


def kernel(reps):
    m, n = reps.shape
    rows = 128
    mask, binary = pl.pallas_call(
        _mask_kernel,
        grid=(m // rows,),
        in_specs=[pl.BlockSpec((rows, n), lambda i: (i, 0))],
        out_specs=[
            pl.BlockSpec((rows, n), lambda i: (i, 0)),
            pl.BlockSpec((rows, n), lambda i: (i, 0)),
        ],
        out_shape=[
            jax.ShapeDtypeStruct((m, n), jnp.float32),
            jax.ShapeDtypeStruct((m, n), jnp.bfloat16),
        ],
    )(reps)

    bm = 512
    overlaps = pl.pallas_call(
        _overlap_kernel,
        grid=(m // bm, m // bm),
        in_specs=[
            pl.BlockSpec((bm, n), lambda i, j: (i, 0)),
            pl.BlockSpec((bm, n), lambda i, j: (j, 0)),
        ],
        out_specs=pl.BlockSpec((bm, bm), lambda i, j: (i, j)),
        out_shape=jax.ShapeDtypeStruct((m, m), jnp.float32),
    )(binary, binary)
    return (mask, overlaps)


# speedup vs baseline: 1.4451x; 1.0160x over previous
import jax
import jax.numpy as jnp
from jax.experimental import pallas as pl

_K = 128
_INT_MIN = -2147483648
_INT_MAX = 2147483647


def _mask_kernel(reps_ref, mask_ref, bin_ref):
    x = reps_ref[...]
    rows, n = x.shape
    b = jax.lax.bitcast_convert_type(x, jnp.int32)
    key = jnp.where(b >= 0, b, (~b) ^ jnp.int32(_INT_MIN))

    def hybrid_step(i, state):
        lo, hi, done, thr, f_lo, f_hi, side = state
        lo_f = lo.astype(jnp.float32)
        span = hi.astype(jnp.float32) + 1.0 - lo_f
        frac = f_lo / jnp.maximum(f_lo - f_hi, 1.0)
        mid_f = jnp.clip(lo_f + frac * span, -2.14e9, 2.14e9)
        mid = jnp.clip(mid_f.astype(jnp.int32), lo + 1, hi)
        cnt = jnp.sum((key >= mid).astype(jnp.int32), axis=1, keepdims=True)
        res = cnt.astype(jnp.float32) - jnp.float32(_K)
        ge = cnt >= _K
        hit = (cnt == _K) & (done < 1)
        thr = jnp.where(hit, mid, thr)
        done = jnp.where(hit, jnp.int32(1), done)
        lo = jnp.where(ge, mid, lo)
        hi = jnp.where(ge, hi, mid - 1)
        stall_hi = jnp.logical_not(ge) & (side < 0)
        stall_lo = ge & (side > 0)
        f_lo = jnp.where(ge, res, jnp.where(stall_hi, f_lo * 0.5, f_lo))
        f_hi = jnp.where(ge, jnp.where(stall_lo, f_hi * 0.5, f_hi), res)
        side = jnp.where(ge, jnp.int32(1), jnp.int32(-1))
        return lo, hi, done, thr, f_lo, f_hi, side

    lo0 = jnp.min(key, axis=1, keepdims=True)
    hi0 = jnp.max(key, axis=1, keepdims=True)
    done0 = jnp.zeros((rows, 1), jnp.int32)
    thr0 = jnp.zeros((rows, 1), jnp.int32)
    flo0 = jnp.full((rows, 1), float(n - _K), jnp.float32)
    fhi0 = jnp.full((rows, 1), float(-_K), jnp.float32)
    side0 = jnp.zeros((rows, 1), jnp.int32)
    lo, hi, done, thr, _, _, _ = jax.lax.fori_loop(
        0, 13, hybrid_step, (lo0, hi0, done0, thr0, flo0, fhi0, side0))

    def mop_cond(state):
        i, lo, hi, done = state[0], state[1], state[2], state[3]
        resolved = (done > 0) | (lo >= hi)
        return (i < 40) & jnp.logical_not(jnp.all(resolved))

    def mop_step(state):
        i, lo, hi, done, thr = state
        x_xor = lo ^ hi
        mid = (lo & hi) + (x_xor >> 1) + (x_xor & 1)
        cnt = jnp.sum((key >= mid).astype(jnp.int32), axis=1, keepdims=True)
        ge = cnt >= _K
        hit = (cnt == _K) & (done < 1)
        thr = jnp.where(hit, mid, thr)
        done = jnp.where(hit, jnp.int32(1), done)
        lo = jnp.where(ge, mid, lo)
        hi = jnp.where(ge, hi, mid - 1)
        return i + 1, lo, hi, done, thr

    _, lo, hi, done, thr = jax.lax.while_loop(
        mop_cond, mop_step, (jnp.int32(0), lo, hi, done, thr))
    done = done > 0

    on_clean = (key >= thr).astype(jnp.float32)

    def tie_path(_):
        t_exact = lo
        c_gt = jnp.sum((key > t_exact).astype(jnp.int32), axis=1,
                       keepdims=True)
        need = _K - c_gt

        def index_step(_, state):
            lo_i, hi_i = state
            mid = (lo_i & hi_i) + ((lo_i ^ hi_i) >> 1)
            iota = jax.lax.broadcasted_iota(jnp.int32, (rows, n), 1)
            sel = (key == t_exact) & (iota <= mid)
            cnt = jnp.sum(sel.astype(jnp.int32), axis=1, keepdims=True)
            ge = cnt >= need
            return jnp.where(ge, lo_i, mid + 1), jnp.where(ge, mid, hi_i)

        lo0i = jnp.zeros((rows, 1), jnp.int32)
        hi0i = jnp.full((rows, 1), n - 1, jnp.int32)
        cut, _ = jax.lax.fori_loop(0, 13, index_step, (lo0i, hi0i))
        iota = jax.lax.broadcasted_iota(jnp.int32, (rows, n), 1)
        on_tie = ((key > t_exact)
                  | ((key == t_exact) & (iota <= cut))).astype(jnp.float32)
        return jnp.where(done, on_clean, on_tie)

    on = jax.lax.cond(jnp.all(done),
                      lambda _: on_clean,
                      tie_path,
                      None)
    mask_ref[...] = on * jnp.float32(2.0) - jnp.float32(1.0)
    bin_ref[...] = on.astype(jnp.bfloat16)


def _overlap_kernel(bi_ref, bj_ref, out_ref):
    k = pl.program_id(2)

    @pl.when(k == 0)
    def _init():
        out_ref[...] = jnp.zeros_like(out_ref)

    out_ref[...] += jax.lax.dot_general(
        bi_ref[...], bj_ref[...], (((1,), (1,)), ((), ())),
        preferred_element_type=jnp.float32)


def kernel(reps):
    m, n = reps.shape
    rows = 128
    mask, binary = pl.pallas_call(
        _mask_kernel,
        grid=(m // rows,),
        in_specs=[pl.BlockSpec((rows, n), lambda i: (i, 0))],
        out_specs=[
            pl.BlockSpec((rows, n), lambda i: (i, 0)),
            pl.BlockSpec((rows, n), lambda i: (i, 0)),
        ],
        out_shape=[
            jax.ShapeDtypeStruct((m, n), jnp.float32),
            jax.ShapeDtypeStruct((m, n), jnp.bfloat16),
        ],
    )(reps)

    bm = 1024
    bk = 4096
    overlaps = pl.pallas_call(
        _overlap_kernel,
        grid=(m // bm, m // bm, n // bk),
        in_specs=[
            pl.BlockSpec((bm, bk), lambda i, j, k: (i, k)),
            pl.BlockSpec((bm, bk), lambda i, j, k: (j, k)),
        ],
        out_specs=pl.BlockSpec((bm, bm), lambda i, j, k: (i, j)),
        out_shape=jax.ShapeDtypeStruct((m, m), jnp.float32),
    )(binary, binary)
    return (mask, overlaps)
